# Initial kernel scaffold; baseline (speedup 1.0000x reference)
import jax
import jax.numpy as jnp
from jax.experimental import pallas as pl


def kernel(node_feat, relation_feat, relation_weights, self_weights, weight):
    raise NotImplementedError("write your pallas kernel here")


# speedup vs baseline: 2.5712x; 2.5712x over previous
import jax
import jax.numpy as jnp
from jax.experimental import pallas as pl
from jax.experimental.pallas import tpu as pltpu

B = 10000
D = 128
O = 128
R = 8
NEIGH = 8
BM = 400


def _rgcn_block(node_ref, x_ref, rw_ref, sw_ref, w_ref, out_ref, wc_ref):
    @pl.when(pl.program_id(0) == 0)
    def _fold():
        for r in range(R):
            wc_ref[r] = jnp.dot(
                rw_ref[r], w_ref[r * O:(r + 1) * O, :],
                preferred_element_type=jnp.float32) * (1.0 / NEIGH)
        wc_ref[R] = jnp.dot(
            sw_ref[...], w_ref[R * O:(R + 1) * O, :],
            preferred_element_type=jnp.float32)

    acc = jnp.dot(node_ref[...], wc_ref[R], preferred_element_type=jnp.float32)
    for r in range(R):
        base = r * NEIGH * D
        s = x_ref[:, base:base + D]
        for n in range(1, NEIGH):
            s = s + x_ref[:, base + n * D:base + (n + 1) * D]
        acc = acc + jnp.dot(s, wc_ref[r], preferred_element_type=jnp.float32)
    out_ref[...] = jnp.maximum(acc, 0.0)


def kernel(node_feat, relation_feat, relation_weights, self_weights, weight):
    grid = (B // BM,)
    return pl.pallas_call(
        _rgcn_block,
        grid=grid,
        in_specs=[
            pl.BlockSpec((BM, D), lambda i: (i, 0)),
            pl.BlockSpec((BM, R * NEIGH * D), lambda i: (i, 0)),
            pl.BlockSpec((R, D, O), lambda i: (0, 0, 0)),
            pl.BlockSpec((D, O), lambda i: (0, 0)),
            pl.BlockSpec(((R + 1) * O, O), lambda i: (0, 0)),
        ],
        out_specs=pl.BlockSpec((BM, O), lambda i: (i, 0)),
        out_shape=jax.ShapeDtypeStruct((B, O), jnp.float32),
        scratch_shapes=[pltpu.VMEM((R + 1, D, O), jnp.float32)],
        compiler_params=pltpu.CompilerParams(
            dimension_semantics=("arbitrary",)),
    )(node_feat, relation_feat, relation_weights, self_weights, weight)


# speedup vs baseline: 2.6198x; 1.0189x over previous
import jax
import jax.numpy as jnp
from jax.experimental import pallas as pl
from jax.experimental.pallas import tpu as pltpu

B = 10000
D = 128
O = 128
R = 8
NEIGH = 8
BM = 200


def _rgcn_block(node_ref, x_ref, rw_ref, sw_ref, w_ref, out_ref, wc_ref):
    @pl.when(pl.program_id(0) == 0)
    def _fold():
        for r in range(R):
            wc_ref[r] = jnp.dot(
                rw_ref[r], w_ref[r * O:(r + 1) * O, :],
                preferred_element_type=jnp.float32) * (1.0 / NEIGH)
        wc_ref[R] = jnp.dot(
            sw_ref[...], w_ref[R * O:(R + 1) * O, :],
            preferred_element_type=jnp.float32)

    acc = jnp.dot(node_ref[...], wc_ref[R], preferred_element_type=jnp.float32)
    for r in range(R):
        base = r * NEIGH * D
        s = x_ref[:, base:base + D]
        for n in range(1, NEIGH):
            s = s + x_ref[:, base + n * D:base + (n + 1) * D]
        acc = acc + jnp.dot(s, wc_ref[r], preferred_element_type=jnp.float32)
    out_ref[...] = jnp.maximum(acc, 0.0)


def kernel(node_feat, relation_feat, relation_weights, self_weights, weight):
    grid = (B // BM,)
    return pl.pallas_call(
        _rgcn_block,
        grid=grid,
        in_specs=[
            pl.BlockSpec((BM, D), lambda i: (i, 0)),
            pl.BlockSpec((BM, R * NEIGH * D), lambda i: (i, 0)),
            pl.BlockSpec((R, D, O), lambda i: (0, 0, 0)),
            pl.BlockSpec((D, O), lambda i: (0, 0)),
            pl.BlockSpec(((R + 1) * O, O), lambda i: (0, 0)),
        ],
        out_specs=pl.BlockSpec((BM, O), lambda i: (i, 0)),
        out_shape=jax.ShapeDtypeStruct((B, O), jnp.float32),
        scratch_shapes=[pltpu.VMEM((R + 1, D, O), jnp.float32)],
        compiler_params=pltpu.CompilerParams(
            dimension_semantics=("arbitrary",)),
    )(node_feat, relation_feat, relation_weights, self_weights, weight)
